# Initial kernel scaffold; baseline (speedup 1.0000x reference)
import jax
import jax.numpy as jnp
from jax.experimental import pallas as pl


def kernel(xyz2, xyz1):
    raise NotImplementedError("write your pallas kernel here")


# speedup vs baseline: 10.4834x; 10.4834x over previous
import functools

import jax
import jax.numpy as jnp
from jax import lax
from jax.experimental import pallas as pl
from jax.experimental.pallas import tpu as pltpu

K = 16
QB = 256


def _body(q_ref, r_ref, out_ref):
    q = q_ref[0]
    r = r_ref[0]
    n2 = r.shape[0]

    qn = jnp.sum(q * q, axis=1)
    rn = jnp.sum(r * r, axis=1)
    qb = q.astype(jnp.bfloat16).astype(jnp.float32)
    rb = r.astype(jnp.bfloat16).astype(jnp.float32)
    cross = (qb[:, 0:1] * rb[:, 0][None, :]
             + qb[:, 1:2] * rb[:, 1][None, :]
             + qb[:, 2:3] * rb[:, 2][None, :])
    d = (qn[:, None] + rn[None, :]) - 2.0 * cross
    d = jnp.where(jnp.isnan(d), 0.0, d)
    d = jnp.maximum(d, 0.0)

    iota = lax.broadcasted_iota(jnp.int32, (QB, n2), 1)
    inf = jnp.float32(jnp.inf)
    for k in range(K + 1):
        m = jnp.min(d, axis=1)
        cand = jnp.where(d == m[:, None], iota, n2)
        idx = jnp.min(cand, axis=1).astype(jnp.int32)
        if k > 0:
            out_ref[0, k - 1, :] = idx
        if k < K:
            d = jnp.where(iota == idx[:, None], inf, d)


@functools.partial(jax.jit)
def _run(xyz2, xyz1):
    b, n1, _ = xyz1.shape
    n2 = xyz2.shape[1]
    grid = (b, n1 // QB)
    out = pl.pallas_call(
        _body,
        grid=grid,
        in_specs=[
            pl.BlockSpec((1, QB, 3), lambda bi, qi: (bi, qi, 0)),
            pl.BlockSpec((1, n2, 3), lambda bi, qi: (bi, 0, 0)),
        ],
        out_specs=pl.BlockSpec((1, K, QB), lambda bi, qi: (bi, 0, qi)),
        out_shape=jax.ShapeDtypeStruct((b, K, n1), jnp.int32),
    )(xyz1, xyz2)
    return jnp.transpose(out, (0, 2, 1))


def kernel(xyz2, xyz1):
    return _run(xyz2, xyz1)
